# Initial kernel scaffold; baseline (speedup 1.0000x reference)
import jax
import jax.numpy as jnp
from jax.experimental import pallas as pl


def kernel(flux, tile_idx, fib_idx, eta):
    raise NotImplementedError("write your pallas kernel here")


# speedup vs baseline: 17.4045x; 17.4045x over previous
import jax
import jax.numpy as jnp
from jax import lax
from jax.experimental import pallas as pl
from jax.experimental.pallas import tpu as pltpu
from jax.experimental.pallas import tpu_sc as plsc

_N_TILES = 8
_N_FIBRES = 5000
_NB = _N_TILES * _N_FIBRES
_DUMMY = _NB
_NB_PAD = 40160
_ROWS = 128
_COLS = 20000
_L = 16
_N_WORKERS = 32
_ROWS_PER_W = _ROWS // _N_WORKERS
_UNROLL = 10
_B_LEFT = 1 << 16
_B_RIGHT = 1 << 17


def _sc_body(pk_hbm, flux_hbm, w_hbm, out_hbm, pk_v, row_v, acc_v, w_v):
    cid = lax.axis_index("c")
    sid = lax.axis_index("s")
    wid = sid * 2 + cid

    pltpu.sync_copy(w_hbm, w_v)
    pltpu.sync_copy(pk_hbm, pk_v)
    w0 = w_v[pl.ds(0, _L)]
    eta = w_v[pl.ds(_L, _L)]
    zeros = jnp.zeros((_L,), jnp.float32)
    dummy = jnp.full((_L,), _DUMMY, jnp.int32)
    kmask = jnp.full((_L,), 0xFFFF, jnp.int32)
    blft = jnp.full((_L,), _B_LEFT, jnp.int32)
    brgt = jnp.full((_L,), _B_RIGHT, jnp.int32)
    zi = jnp.zeros((_L,), jnp.int32)

    @plsc.parallel_loop(0, _NB_PAD, step=_L, unroll=_UNROLL)
    def _zero(o):
        acc_v[pl.ds(o, _L)] = zeros

    for rr in range(_ROWS_PER_W):
        r = wid * _ROWS_PER_W + rr
        pltpu.sync_copy(flux_hbm.at[r], row_v)

        @plsc.parallel_loop(0, _COLS, step=_L, unroll=_UNROLL)
        def _scat(o):
            kv = pk_v[pl.ds(o, _L)] & kmask
            fv = row_v[pl.ds(o, _L)]
            plsc.addupdate_scatter(acc_v, [kv], fv)

        @plsc.parallel_loop(0, _COLS, step=_L, unroll=_UNROLL)
        def _gath(o):
            pv = pk_v[pl.ds(o, _L)]
            kv = pv & kmask
            klv = jnp.where((pv & blft) != zi, dummy, kv - 1)
            krv = jnp.where((pv & brgt) != zi, dummy, kv + 1)
            cen = plsc.load_gather(acc_v, [kv])
            lft = plsc.load_gather(acc_v, [klv])
            rgt = plsc.load_gather(acc_v, [krv])
            row_v[pl.ds(o, _L)] = w0 * cen + eta * (lft + rgt)

        pltpu.sync_copy(row_v, out_hbm.at[r])
        if rr != _ROWS_PER_W - 1:
            @plsc.parallel_loop(0, _COLS, step=_L, unroll=_UNROLL)
            def _clear(o):
                kv = pk_v[pl.ds(o, _L)] & kmask
                plsc.store_scatter(acc_v, [kv], zeros)


def kernel(flux, tile_idx, fib_idx, eta):
    input_shape = flux.shape
    flux2 = flux.reshape(-1, flux.shape[-1]) if flux.ndim > 1 else flux[None, :]
    fib = fib_idx.astype(jnp.int32)
    key = tile_idx.astype(jnp.int32) * _N_FIBRES + fib
    pk = (key
          | jnp.where(fib == 0, _B_LEFT, 0)
          | jnp.where(fib == _N_FIBRES - 1, _B_RIGHT, 0))
    eta32 = jnp.asarray(eta, jnp.float32)
    w = jnp.concatenate([
        jnp.full((_L,), 1.0, jnp.float32) - 2.0 * eta32,
        jnp.zeros((_L,), jnp.float32) + eta32,
    ])

    mesh = plsc.VectorSubcoreMesh(core_axis_name="c", subcore_axis_name="s")
    out = pl.kernel(
        _sc_body,
        mesh=mesh,
        compiler_params=pltpu.CompilerParams(needs_layout_passes=False),
        out_type=jax.ShapeDtypeStruct((_ROWS, _COLS), jnp.float32),
        scratch_types=[
            pltpu.VMEM((_COLS,), jnp.int32),
            pltpu.VMEM((_COLS,), jnp.float32),
            pltpu.VMEM((_NB_PAD,), jnp.float32),
            pltpu.VMEM((2 * _L,), jnp.float32),
        ],
    )(pk, flux2, w)
    return out.reshape(input_shape)
